# Initial kernel scaffold; baseline (speedup 1.0000x reference)
import jax
import jax.numpy as jnp
from jax.experimental import pallas as pl


def kernel(input_embeds, position_ids, atom_ids, pos_table, atom_table, ln_gamma, ln_beta):
    raise NotImplementedError("write your pallas kernel here")


# speedup vs baseline: 1.5392x; 1.5392x over previous
import functools

import jax
import jax.numpy as jnp
from jax import lax
from jax.experimental import pallas as pl
from jax.experimental.pallas import tpu as pltpu
from jax.experimental.pallas import tpu_sc as plsc

_B, _S, _D = 4, 4096, 1024
_N = _B * _S
_NC, _NS = 2, 16
_NW = _NC * _NS
_PER_W = _N // _NW
_CHUNK = 64
_NCHUNK = _PER_W // _CHUNK
_EPS = 1e-12


def _sc_gather_both(pos_table, atom_table, pos_ids, atom_ids):
    mesh = plsc.VectorSubcoreMesh(core_axis_name="c", subcore_axis_name="s")
    out_t = [jax.ShapeDtypeStruct((_N, _D), jnp.float32)] * 2

    @functools.partial(
        pl.kernel,
        mesh=mesh,
        out_type=out_t,
        scratch_types=[
            pltpu.VMEM((_PER_W,), jnp.int32),
            pltpu.VMEM((_PER_W,), jnp.int32),
            pltpu.VMEM((_CHUNK, _D), jnp.float32),
            pltpu.SemaphoreType.DMA,
        ],
    )
    def k(pos_tab, atom_tab, pid, aid, pos_out, atom_out, pidx_v, aidx_v, rows_v, sem):
        wid = lax.axis_index("s") * _NC + lax.axis_index("c")
        base = wid * _PER_W
        pltpu.sync_copy(pid.at[pl.ds(base, _PER_W)], pidx_v)
        pltpu.sync_copy(aid.at[pl.ds(base, _PER_W)], aidx_v)

        @pl.loop(0, _NCHUNK)
        def _(c):
            off = c * _CHUNK
            pltpu.async_copy(pos_tab.at[pidx_v.at[pl.ds(off, _CHUNK)]], rows_v, sem).wait()
            pltpu.sync_copy(rows_v, pos_out.at[pl.ds(base + off, _CHUNK)])
            pltpu.async_copy(atom_tab.at[aidx_v.at[pl.ds(off, _CHUNK)]], rows_v, sem).wait()
            pltpu.sync_copy(rows_v, atom_out.at[pl.ds(base + off, _CHUNK)])

    return k(pos_table, atom_table, pos_ids, atom_ids)


def _tc_add_ln(x, pos_emb, atom_emb, gamma, beta):
    rows = 256
    grid = (_N // rows,)

    def body(x_ref, p_ref, a_ref, g_ref, b_ref, o_ref):
        v = x_ref[...] + p_ref[...] + a_ref[...]
        mean = jnp.mean(v, axis=-1, keepdims=True)
        vc = v - mean
        var = jnp.mean(vc * vc, axis=-1, keepdims=True)
        o_ref[...] = vc * lax.rsqrt(var + _EPS) * g_ref[...] + b_ref[...]

    row_spec = pl.BlockSpec((rows, _D), lambda i: (i, 0))
    vec_spec = pl.BlockSpec((1, _D), lambda i: (0, 0))
    return pl.pallas_call(
        body,
        grid=grid,
        in_specs=[row_spec, row_spec, row_spec, vec_spec, vec_spec],
        out_specs=row_spec,
        out_shape=jax.ShapeDtypeStruct((_N, _D), jnp.float32),
    )(x, pos_emb, atom_emb, gamma, beta)


def kernel(input_embeds, position_ids, atom_ids, pos_table, atom_table, ln_gamma, ln_beta):
    pid = position_ids.reshape(-1).astype(jnp.int32)
    aid = atom_ids.reshape(-1).astype(jnp.int32)
    pos_emb, atom_emb = _sc_gather_both(pos_table, atom_table, pid, aid)
    out = _tc_add_ln(
        input_embeds.reshape(_N, _D),
        pos_emb,
        atom_emb,
        ln_gamma.reshape(1, _D),
        ln_beta.reshape(1, _D),
    )
    return out.reshape(_B, _S, _D)


# speedup vs baseline: 1.6046x; 1.0425x over previous
import functools

import jax
import jax.numpy as jnp
from jax import lax
from jax.experimental import pallas as pl
from jax.experimental.pallas import tpu as pltpu
from jax.experimental.pallas import tpu_sc as plsc

_B, _S, _D = 4, 4096, 1024
_N = _B * _S
_NC, _NS = 2, 16
_NW = _NC * _NS
_K = 4
_NTOK = _N // _K
_PER_W = _NTOK // _NW
_CHUNK = 64
_NCHUNK = _PER_W // _CHUNK
_ROWS = 512
_NB = _NTOK // _ROWS
_EPS = 1e-12


def _sc_gather_chunk(j):
    mesh = plsc.VectorSubcoreMesh(core_axis_name="c", subcore_axis_name="s")
    out_t = [jax.ShapeDtypeStruct((_NTOK, _D), jnp.float32)] * 2
    cbase = j * _NTOK

    @functools.partial(
        pl.kernel,
        mesh=mesh,
        out_type=out_t,
        scratch_types=[
            pltpu.VMEM((_PER_W,), jnp.int32),
            pltpu.VMEM((_PER_W,), jnp.int32),
            pltpu.VMEM((_CHUNK, _D), jnp.float32),
            pltpu.SemaphoreType.DMA,
        ],
    )
    def k(pos_tab, atom_tab, pid, aid, pos_out, atom_out, pidx_v, aidx_v, rows_v, sem):
        wid = lax.axis_index("s") * _NC + lax.axis_index("c")
        base = wid * _PER_W
        pltpu.sync_copy(pid.at[pl.ds(cbase + base, _PER_W)], pidx_v)
        pltpu.sync_copy(aid.at[pl.ds(cbase + base, _PER_W)], aidx_v)

        @pl.loop(0, _NCHUNK)
        def _(c):
            off = c * _CHUNK
            pltpu.async_copy(pos_tab.at[pidx_v.at[pl.ds(off, _CHUNK)]], rows_v, sem).wait()
            pltpu.sync_copy(rows_v, pos_out.at[pl.ds(base + off, _CHUNK)])
            pltpu.async_copy(atom_tab.at[aidx_v.at[pl.ds(off, _CHUNK)]], rows_v, sem).wait()
            pltpu.sync_copy(rows_v, atom_out.at[pl.ds(base + off, _CHUNK)])

    return k


def _addln_body(x_ref, p_ref, a_ref, g_ref, b_ref, o_ref):
    v = x_ref[...] + p_ref[...] + a_ref[...]
    mean = jnp.mean(v, axis=-1, keepdims=True)
    vc = v - mean
    var = jnp.mean(vc * vc, axis=-1, keepdims=True)
    o_ref[...] = vc * lax.rsqrt(var + _EPS) * g_ref[...] + b_ref[...]


def _tc_chunk(j, buf, x, pos_j, atom_j, gamma, beta):
    row_spec = pl.BlockSpec((_ROWS, _D), lambda i, j=j: (j * _NB + i, 0))
    chunk_spec = pl.BlockSpec((_ROWS, _D), lambda i: (i, 0))
    vec_spec = pl.BlockSpec((1, _D), lambda i: (0, 0))
    common = dict(
        grid=(_NB,),
        out_specs=row_spec,
        out_shape=jax.ShapeDtypeStruct((_N, _D), jnp.float32),
    )
    if buf is None:
        return pl.pallas_call(
            _addln_body,
            in_specs=[row_spec, chunk_spec, chunk_spec, vec_spec, vec_spec],
            **common,
        )(x, pos_j, atom_j, gamma, beta)

    def body(buf_ref, x_ref, p_ref, a_ref, g_ref, b_ref, o_ref):
        _addln_body(x_ref, p_ref, a_ref, g_ref, b_ref, o_ref)

    return pl.pallas_call(
        body,
        in_specs=[
            pl.BlockSpec(memory_space=pl.ANY),
            row_spec,
            chunk_spec,
            chunk_spec,
            vec_spec,
            vec_spec,
        ],
        input_output_aliases={0: 0},
        **common,
    )(buf, x, pos_j, atom_j, gamma, beta)


def kernel(input_embeds, position_ids, atom_ids, pos_table, atom_table, ln_gamma, ln_beta):
    pid = position_ids.reshape(-1).astype(jnp.int32)
    aid = atom_ids.reshape(-1).astype(jnp.int32)
    x = input_embeds.reshape(_N, _D)
    gamma = ln_gamma.reshape(1, _D)
    beta = ln_beta.reshape(1, _D)

    gathered = [_sc_gather_chunk(j)(pos_table, atom_table, pid, aid) for j in range(_K)]
    buf = None
    for j, (pos_j, atom_j) in enumerate(gathered):
        buf = _tc_chunk(j, buf, x, pos_j, atom_j, gamma, beta)
    return buf.reshape(_B, _S, _D)


# speedup vs baseline: 1.6420x; 1.0233x over previous
import functools

import jax
import jax.numpy as jnp
from jax import lax
from jax.experimental import pallas as pl
from jax.experimental.pallas import tpu as pltpu
from jax.experimental.pallas import tpu_sc as plsc

_B, _S, _D = 4, 4096, 1024
_N = _B * _S
_NC, _NS = 2, 16
_NW = _NC * _NS
_K = 4
_NTOK = _N // _K
_PER_W = _NTOK // _NW
_GC = 16
_NBUF = 4
_NITEM = 2 * (_PER_W // _GC)
_ROWS = 512
_NB = _NTOK // _ROWS
_EPS = 1e-12


def _sc_gather_chunk(j):
    mesh = plsc.VectorSubcoreMesh(core_axis_name="c", subcore_axis_name="s")
    out_t = [jax.ShapeDtypeStruct((_NTOK, _D), jnp.float32)] * 2
    cbase = j * _NTOK

    @functools.partial(
        pl.kernel,
        mesh=mesh,
        out_type=out_t,
        scratch_types=[
            pltpu.VMEM((_PER_W,), jnp.int32),
            pltpu.VMEM((_PER_W,), jnp.int32),
        ]
        + [pltpu.VMEM((_GC, _D), jnp.float32)] * _NBUF
        + [pltpu.SemaphoreType.DMA] * (2 * _NBUF),
    )
    def k(pos_tab, atom_tab, pid, aid, pos_out, atom_out, pidx_v, aidx_v, *scr):
        bufs = scr[:_NBUF]
        gsem = scr[_NBUF : 2 * _NBUF]
        wsem = scr[2 * _NBUF :]
        wid = lax.axis_index("s") * _NC + lax.axis_index("c")
        base = wid * _PER_W
        pltpu.sync_copy(pid.at[pl.ds(cbase + base, _PER_W)], pidx_v)
        pltpu.sync_copy(aid.at[pl.ds(cbase + base, _PER_W)], aidx_v)

        def gather(k):
            b = k % _NBUF
            tab, idx = (pos_tab, pidx_v) if k % 2 == 0 else (atom_tab, aidx_v)
            off = (k // 2) * _GC
            pltpu.make_async_copy(
                tab.at[idx.at[pl.ds(off, _GC)]], bufs[b], gsem[b]
            ).start()

        def gwait(k):
            b = k % _NBUF
            tab, idx = (pos_tab, pidx_v) if k % 2 == 0 else (atom_tab, aidx_v)
            off = (k // 2) * _GC
            pltpu.make_async_copy(
                tab.at[idx.at[pl.ds(off, _GC)]], bufs[b], gsem[b]
            ).wait()

        def _wcopy(k):
            b = k % _NBUF
            out = pos_out if k % 2 == 0 else atom_out
            off = (k // 2) * _GC
            return pltpu.make_async_copy(
                bufs[b], out.at[pl.ds(base + off, _GC)], wsem[b]
            )

        for k in range(_NITEM):
            if k >= _NBUF:
                _wcopy(k - _NBUF).wait()
            gather(k)
            if k >= 1:
                gwait(k - 1)
                _wcopy(k - 1).start()
        gwait(_NITEM - 1)
        _wcopy(_NITEM - 1).start()
        for k in range(_NITEM - _NBUF + 1, _NITEM):
            _wcopy(k).wait()

    return k


def _addln_body(x_ref, p_ref, a_ref, g_ref, b_ref, o_ref):
    v = x_ref[...] + p_ref[...] + a_ref[...]
    mean = jnp.mean(v, axis=-1, keepdims=True)
    vc = v - mean
    var = jnp.mean(vc * vc, axis=-1, keepdims=True)
    o_ref[...] = vc * lax.rsqrt(var + _EPS) * g_ref[...] + b_ref[...]


def _tc_chunk(j, buf, x, pos_j, atom_j, gamma, beta):
    row_spec = pl.BlockSpec((_ROWS, _D), lambda i, j=j: (j * _NB + i, 0))
    chunk_spec = pl.BlockSpec((_ROWS, _D), lambda i: (i, 0))
    vec_spec = pl.BlockSpec((1, _D), lambda i: (0, 0))
    common = dict(
        grid=(_NB,),
        out_specs=row_spec,
        out_shape=jax.ShapeDtypeStruct((_N, _D), jnp.float32),
    )
    if buf is None:
        return pl.pallas_call(
            _addln_body,
            in_specs=[row_spec, chunk_spec, chunk_spec, vec_spec, vec_spec],
            **common,
        )(x, pos_j, atom_j, gamma, beta)

    def body(buf_ref, x_ref, p_ref, a_ref, g_ref, b_ref, o_ref):
        _addln_body(x_ref, p_ref, a_ref, g_ref, b_ref, o_ref)

    return pl.pallas_call(
        body,
        in_specs=[
            pl.BlockSpec(memory_space=pl.ANY),
            row_spec,
            chunk_spec,
            chunk_spec,
            vec_spec,
            vec_spec,
        ],
        input_output_aliases={0: 0},
        **common,
    )(buf, x, pos_j, atom_j, gamma, beta)


def kernel(input_embeds, position_ids, atom_ids, pos_table, atom_table, ln_gamma, ln_beta):
    pid = position_ids.reshape(-1).astype(jnp.int32)
    aid = atom_ids.reshape(-1).astype(jnp.int32)
    x = input_embeds.reshape(_N, _D)
    gamma = ln_gamma.reshape(1, _D)
    beta = ln_beta.reshape(1, _D)

    gathered = [_sc_gather_chunk(j)(pos_table, atom_table, pid, aid) for j in range(_K)]
    buf = None
    for j, (pos_j, atom_j) in enumerate(gathered):
        buf = _tc_chunk(j, buf, x, pos_j, atom_j, gamma, beta)
    return buf.reshape(_B, _S, _D)


# speedup vs baseline: 1.8337x; 1.1167x over previous
import functools

import jax
import jax.numpy as jnp
from jax import lax
from jax.experimental import pallas as pl
from jax.experimental.pallas import tpu as pltpu
from jax.experimental.pallas import tpu_sc as plsc

_B, _S, _D = 4, 4096, 1024
_N = _B * _S
_NC, _NS = 2, 16
_NW = _NC * _NS
_K = 4
_NTOK = _N // _K
_PER_W = _NTOK // _NW
_GC = 16
_NG = _PER_W // _GC
_LANES = 16
_ROWS = 512
_NB = _NTOK // _ROWS
_EPS = 1e-12


def _sc_gather_chunk(j):
    mesh = plsc.VectorSubcoreMesh(core_axis_name="c", subcore_axis_name="s")
    cbase = j * _NTOK

    @functools.partial(
        pl.kernel,
        mesh=mesh,
        out_type=jax.ShapeDtypeStruct((_NTOK, _D), jnp.float32),
        scratch_types=[
            pltpu.VMEM((_PER_W,), jnp.int32),
            pltpu.VMEM((_PER_W,), jnp.int32),
        ]
        + [pltpu.VMEM((_GC, _D), jnp.float32)] * 4
        + [pltpu.SemaphoreType.DMA] * 6,
    )
    def k(pos_tab, atom_tab, pid, aid, sum_out, pidx_v, aidx_v, *scr):
        bufp = scr[0:2]
        bufa = scr[2:4]
        gsemp = scr[4:6]
        gsema = scr[6:8]
        wsem = scr[8:10]
        wid = lax.axis_index("s") * _NC + lax.axis_index("c")
        base = wid * _PER_W
        pltpu.sync_copy(pid.at[pl.ds(cbase + base, _PER_W)], pidx_v)
        pltpu.sync_copy(aid.at[pl.ds(cbase + base, _PER_W)], aidx_v)

        def _gcopies(c):
            b = c % 2
            off = c * _GC
            return (
                pltpu.make_async_copy(
                    pos_tab.at[pidx_v.at[pl.ds(off, _GC)]], bufp[b], gsemp[b]
                ),
                pltpu.make_async_copy(
                    atom_tab.at[aidx_v.at[pl.ds(off, _GC)]], bufa[b], gsema[b]
                ),
            )

        def _wcopy(c):
            b = c % 2
            off = c * _GC
            return pltpu.make_async_copy(
                bufp[b], sum_out.at[pl.ds(base + off, _GC)], wsem[b]
            )

        def _add(c):
            b = c % 2
            bp, ba = bufp[b], bufa[b]

            @pl.loop(0, _GC)
            def _(r):
                for u in range(_D // _LANES):
                    sl = pl.ds(u * _LANES, _LANES)
                    bp[r, sl] = bp[r, sl] + ba[r, sl]

        def _process(c):
            for cp in _gcopies(c):
                cp.wait()
            _add(c)
            _wcopy(c).start()

        for c in range(_NG):
            if c >= 2:
                _wcopy(c - 2).wait()
            for cp in _gcopies(c):
                cp.start()
            if c >= 1:
                _process(c - 1)
        _process(_NG - 1)
        _wcopy(_NG - 2).wait()
        _wcopy(_NG - 1).wait()

    return k


def _addln_body(x_ref, s_ref, g_ref, b_ref, o_ref):
    v = x_ref[...] + s_ref[...]
    mean = jnp.mean(v, axis=-1, keepdims=True)
    vc = v - mean
    var = jnp.mean(vc * vc, axis=-1, keepdims=True)
    o_ref[...] = vc * lax.rsqrt(var + _EPS) * g_ref[...] + b_ref[...]


def _tc_chunk(j, buf, x, sum_j, gamma, beta):
    row_spec = pl.BlockSpec((_ROWS, _D), lambda i, j=j: (j * _NB + i, 0))
    chunk_spec = pl.BlockSpec((_ROWS, _D), lambda i: (i, 0))
    vec_spec = pl.BlockSpec((1, _D), lambda i: (0, 0))
    common = dict(
        grid=(_NB,),
        out_specs=row_spec,
        out_shape=jax.ShapeDtypeStruct((_N, _D), jnp.float32),
    )
    if buf is None:
        return pl.pallas_call(
            _addln_body,
            in_specs=[row_spec, chunk_spec, vec_spec, vec_spec],
            **common,
        )(x, sum_j, gamma, beta)

    def body(buf_ref, x_ref, s_ref, g_ref, b_ref, o_ref):
        _addln_body(x_ref, s_ref, g_ref, b_ref, o_ref)

    return pl.pallas_call(
        body,
        in_specs=[
            pl.BlockSpec(memory_space=pl.ANY),
            row_spec,
            chunk_spec,
            vec_spec,
            vec_spec,
        ],
        input_output_aliases={0: 0},
        **common,
    )(buf, x, sum_j, gamma, beta)


def kernel(input_embeds, position_ids, atom_ids, pos_table, atom_table, ln_gamma, ln_beta):
    pid = position_ids.reshape(-1).astype(jnp.int32)
    aid = atom_ids.reshape(-1).astype(jnp.int32)
    x = input_embeds.reshape(_N, _D)
    gamma = ln_gamma.reshape(1, _D)
    beta = ln_beta.reshape(1, _D)

    sums = [_sc_gather_chunk(j)(pos_table, atom_table, pid, aid) for j in range(_K)]
    buf = None
    for j, sum_j in enumerate(sums):
        buf = _tc_chunk(j, buf, x, sum_j, gamma, beta)
    return buf.reshape(_B, _S, _D)
